# baseline (speedup 1.0000x reference)
import functools

import jax
import jax.numpy as jnp
from jax import lax
from jax.experimental import pallas as pl
from jax.experimental.pallas import tpu as pltpu
from jax.experimental.pallas import tpu_sc as plsc

_T = 4
_TH = _TW = 1024
_C = 4
_L = 16
_WL = 128

_P = 1024
_RPB = 4


def _sc_build_table(tex5s):
    info = plsc.get_sparse_core_info()
    mesh = plsc.VectorSubcoreMesh(core_axis_name="c", subcore_axis_name="s")
    blocks = _TH // 8 // _RPB

    @functools.partial(
        pl.kernel,
        mesh=mesh,
        out_type=jax.ShapeDtypeStruct((_T * _TH * _TW, 2 * _C), jnp.float32),
        compiler_params=pltpu.CompilerParams(
            needs_layout_passes=False, use_tc_tiling_on_sc=False),
        scratch_types=[
            pltpu.VMEM((2, _RPB, _TW // _WL, _C, _WL), jnp.float32),
            pltpu.VMEM((2, _RPB * _TW, 2 * _C), jnp.float32),
            pltpu.SemaphoreType.DMA,
            pltpu.SemaphoreType.DMA,
        ],
    )
    def bk(t0, t1, t2, t3, tab, in_v, out_v, sem_in, sem_out):
        wid = lax.axis_index("s") * info.num_cores + lax.axis_index("c")
        m = wid >> 3
        hb = (wid & 7) * (_TH // 8)
        lanes = lax.iota(jnp.int32, _L)

        def build(tex_ref, mm):
            def tab_slice(bi):
                h0 = hb + bi * _RPB
                row0 = pl.multiple_of(mm * (_TH * _TW) + h0 * _TW,
                                      _RPB * _TW)
                return tab.at[pl.ds(row0, _RPB * _TW)]

            def issue_in(bi, s):
                pltpu.async_copy(tex_ref.at[pl.ds(hb + bi * _RPB, _RPB)],
                                 in_v.at[s], sem_in)

            def wait_in(bi, s):
                pltpu.make_async_copy(
                    tex_ref.at[pl.ds(hb + bi * _RPB, _RPB)], in_v.at[s],
                    sem_in).wait()

            def transpose(s):
                @plsc.parallel_loop(0, _RPB * (_TW // _L), unroll=4)
                def _(i):
                    r = i >> 6
                    g = i & 63
                    wt = g >> 3
                    wl0 = (g & 7) << 4
                    p16 = (r << 10) + (g << 4) + lanes
                    pm1 = (r << 10) + (((g << 4) + lanes - 1) & (_TW - 1))
                    for c in range(_C):
                        t = in_v[s, r, wt, c, pl.ds(wl0, _L)]
                        plsc.store_scatter(
                            out_v.at[s],
                            [p16, jnp.full((_L,), c, jnp.int32)], t)
                        plsc.store_scatter(
                            out_v.at[s],
                            [pm1, jnp.full((_L,), c + 4, jnp.int32)], t)

            issue_in(0, 0)

            def blk_body(bi, _):
                s = bi & 1

                @pl.when(bi + 1 < blocks)
                def _():
                    issue_in(bi + 1, 1 - s)

                wait_in(bi, s)

                @pl.when(bi > 1)
                def _():
                    pltpu.make_async_copy(out_v.at[s], tab_slice(bi - 2),
                                          sem_out).wait()

                transpose(s)
                pltpu.async_copy(out_v.at[s], tab_slice(bi), sem_out)
                return ()

            lax.fori_loop(0, blocks, blk_body, ())
            for tail in (blocks - 2, blocks - 1):
                pltpu.make_async_copy(out_v.at[tail & 1], tab_slice(tail),
                                      sem_out).wait()

        for mm, tex_ref in enumerate((t0, t1, t2, t3)):
            @pl.when(m == mm)
            def _(tex_ref=tex_ref, mm=mm):
                build(tex_ref, mm)

    return bk(*tex5s)


def _sc_sample(uv5, fm5, table, b, h, w):
    n = b * h * w
    info = plsc.get_sparse_core_info()
    nw = info.num_cores * info.num_subcores
    per_w = n // nw
    rows_w = per_w // w
    rpc = _P // w
    n_chunks = per_w // _P
    nwt = w // _WL
    mesh = plsc.VectorSubcoreMesh(core_axis_name="c", subcore_axis_name="s")

    @functools.partial(
        pl.kernel,
        mesh=mesh,
        out_type=jax.ShapeDtypeStruct((b, h, nwt, _C, _WL), jnp.float32),
        compiler_params=pltpu.CompilerParams(
            needs_layout_passes=False, use_tc_tiling_on_sc=False),
        scratch_types=[
            pltpu.VMEM((3, rpc, nwt, 2, _WL), jnp.float32),
            pltpu.VMEM((3, nwt, rpc, _WL), jnp.int32),
            pltpu.VMEM((2, 2, _P), jnp.int32),
            pltpu.VMEM((2, _P, 2 * _C), jnp.float32),
            pltpu.VMEM((2, _P, 2 * _C), jnp.float32),
            pltpu.VMEM((2, rpc, nwt, _C, _WL), jnp.float32),
            pltpu.SemaphoreType.DMA,
            pltpu.SemaphoreType.DMA,
            pltpu.SemaphoreType.DMA,
        ],
    )
    def k(uv_hbm, fm_hbm, tab_hbm, out_hbm,
          uv_v, fm_v, idx_v, top_v, bot_v, o_v, sem_in, sem_g, sem_out):
        wid = lax.axis_index("s") * info.num_cores + lax.axis_index("c")
        lanes = lax.iota(jnp.int32, _L)

        def frac(val, scale):
            x = val * scale - 0.5
            xt = x.astype(jnp.int32)
            x0 = jnp.where(x < xt.astype(jnp.float32), xt - 1, xt)
            return x0, x - x0.astype(jnp.float32)

        def coords(kc):
            r0 = wid * rows_w + kc * rpc
            return r0 // h, r0 % h

        def issue_in(kc, s):
            bi, h0 = coords(kc)
            pltpu.async_copy(uv_hbm.at[bi, pl.ds(h0, rpc)], uv_v.at[s],
                             sem_in)
            pltpu.async_copy(fm_hbm.at[bi, h0 // 8, :, pl.ds(h0 % 8, rpc)],
                             fm_v.at[s], sem_in)

        def wait_in(kc, s):
            bi, h0 = coords(kc)
            pltpu.make_async_copy(uv_hbm.at[bi, pl.ds(h0, rpc)], uv_v.at[s],
                                  sem_in).wait()
            pltpu.make_async_copy(fm_hbm.at[bi, h0 // 8, :,
                                            pl.ds(h0 % 8, rpc)],
                                  fm_v.at[s], sem_in).wait()

        def phase2_and_fire(s3, s):
            @plsc.parallel_loop(0, _P // _L, unroll=4)
            def _(i):
                hs = i >> 5
                wt = (i >> 3) & 3
                wl0 = (i & 7) << 4
                csl = pl.ds(wl0, _L)
                uu = uv_v[s3, hs, wt, 0, csl]
                vv = uv_v[s3, hs, wt, 1, csl]
                fm = fm_v[s3, wt, hs, csl]
                x0, _fx = frac(uu, float(_TW))
                y0, _fy = frac(vv, float(_TH))
                base_m = (fm << 20) + (x0 & (_TW - 1))
                sl = pl.ds(i * _L, _L)
                idx_v[s, 0, sl] = base_m + ((y0 & (_TH - 1)) << 10)
                idx_v[s, 1, sl] = base_m + (((y0 + 1) & (_TH - 1)) << 10)

            pltpu.async_copy(tab_hbm.at[idx_v.at[s, 0]], top_v.at[s], sem_g)
            pltpu.async_copy(tab_hbm.at[idx_v.at[s, 1]], bot_v.at[s], sem_g)

        def combine_and_out(kc, s):
            s3 = kc % 3
            pltpu.make_async_copy(tab_hbm.at[idx_v.at[s, 0]], top_v.at[s],
                                  sem_g).wait()
            pltpu.make_async_copy(tab_hbm.at[idx_v.at[s, 1]], bot_v.at[s],
                                  sem_g).wait()

            @pl.when(kc > 1)
            def _():
                bi2, h02 = coords(kc - 2)
                pltpu.make_async_copy(o_v.at[s], out_hbm.at[bi2,
                                                            pl.ds(h02, rpc)],
                                      sem_out).wait()

            @plsc.parallel_loop(0, _P // _L, unroll=2)
            def _(i):
                hs = i >> 5
                wt = (i >> 3) & 3
                wl0 = (i & 7) << 4
                csl = pl.ds(wl0, _L)
                uu = uv_v[s3, hs, wt, 0, csl]
                vv = uv_v[s3, hs, wt, 1, csl]
                _x0, fx = frac(uu, float(_TW))
                _y0, fy = frac(vv, float(_TH))
                omx = 1.0 - fx
                omy = 1.0 - fy
                p16 = i * _L + lanes
                for c in range(_C):
                    cc = jnp.full((_L,), c, jnp.int32)
                    cc1 = jnp.full((_L,), c + 4, jnp.int32)
                    t00 = plsc.load_gather(top_v.at[s], [p16, cc])
                    t01 = plsc.load_gather(top_v.at[s], [p16, cc1])
                    t10 = plsc.load_gather(bot_v.at[s], [p16, cc])
                    t11 = plsc.load_gather(bot_v.at[s], [p16, cc1])
                    top = t00 * omx + t01 * fx
                    bot = t10 * omx + t11 * fx
                    o_v[s, hs, wt, c, csl] = top * omy + bot * fy

            bi, h0 = coords(kc)
            pltpu.async_copy(o_v.at[s], out_hbm.at[bi, pl.ds(h0, rpc)],
                             sem_out)

        issue_in(0, 0)

        def chunk_body(kc, _):
            s = kc & 1
            s3 = kc % 3
            wait_in(kc, s3)

            @pl.when(kc + 1 < n_chunks)
            def _():
                issue_in(kc + 1, (kc + 1) % 3)

            phase2_and_fire(s3, s)

            @pl.when(kc > 0)
            def _():
                combine_and_out(kc - 1, 1 - s)
            return ()

        lax.fori_loop(0, n_chunks, chunk_body, ())
        combine_and_out(n_chunks - 1, (n_chunks - 1) & 1)
        for tail in (n_chunks - 2, n_chunks - 1):
            bi, h0 = coords(tail)
            pltpu.make_async_copy(o_v.at[tail & 1],
                                  out_hbm.at[bi, pl.ds(h0, rpc)],
                                  sem_out).wait()

    return k(uv5, fm5, table)


def kernel(uv, f_mat, tex0, tex1, tex2, tex3):
    b, h, w, _ = uv.shape
    tex5s = [
        t.reshape(_TH, _TW // _WL, _WL, _C).transpose(0, 1, 3, 2)
        for t in (tex0, tex1, tex2, tex3)
    ]
    table = _sc_build_table(tex5s)
    uv5 = uv.reshape(b, h, w // _WL, _WL, 2).transpose(0, 1, 2, 4, 3)
    fm5 = f_mat.reshape(b, h // 8, 8, w // _WL, _WL).transpose(0, 1, 3, 2, 4)
    out5 = _sc_sample(uv5, fm5, table, b, h, w)
    return out5.transpose(0, 1, 2, 4, 3).reshape(b, h, w, _C)


# speedup vs baseline: 1.0504x; 1.0504x over previous
import functools

import jax
import jax.numpy as jnp
from jax import lax
from jax.experimental import pallas as pl
from jax.experimental.pallas import tpu as pltpu
from jax.experimental.pallas import tpu_sc as plsc

_T = 4
_TH = _TW = 1024
_C = 4
_L = 16
_WL = 128

_P = 2048
_RPB = 4


def _sc_build_table(tex5s):
    info = plsc.get_sparse_core_info()
    mesh = plsc.VectorSubcoreMesh(core_axis_name="c", subcore_axis_name="s")
    blocks = _TH // 8 // _RPB

    @functools.partial(
        pl.kernel,
        mesh=mesh,
        out_type=jax.ShapeDtypeStruct((_T * _TH * _TW, 2 * _C), jnp.float32),
        compiler_params=pltpu.CompilerParams(
            needs_layout_passes=False, use_tc_tiling_on_sc=False),
        scratch_types=[
            pltpu.VMEM((2, _RPB, _TW // _WL, _C, _WL), jnp.float32),
            pltpu.VMEM((2, _RPB * _TW, 2 * _C), jnp.float32),
            pltpu.SemaphoreType.DMA,
            pltpu.SemaphoreType.DMA,
        ],
    )
    def bk(t0, t1, t2, t3, tab, in_v, out_v, sem_in, sem_out):
        wid = lax.axis_index("s") * info.num_cores + lax.axis_index("c")
        m = wid >> 3
        hb = (wid & 7) * (_TH // 8)
        lanes = lax.iota(jnp.int32, _L)

        def build(tex_ref, mm):
            def tab_slice(bi):
                h0 = hb + bi * _RPB
                row0 = pl.multiple_of(mm * (_TH * _TW) + h0 * _TW,
                                      _RPB * _TW)
                return tab.at[pl.ds(row0, _RPB * _TW)]

            def issue_in(bi, s):
                pltpu.async_copy(tex_ref.at[pl.ds(hb + bi * _RPB, _RPB)],
                                 in_v.at[s], sem_in)

            def wait_in(bi, s):
                pltpu.make_async_copy(
                    tex_ref.at[pl.ds(hb + bi * _RPB, _RPB)], in_v.at[s],
                    sem_in).wait()

            def transpose(s):
                @plsc.parallel_loop(0, _RPB * (_TW // _L), unroll=4)
                def _(i):
                    r = i >> 6
                    g = i & 63
                    wt = g >> 3
                    wl0 = (g & 7) << 4
                    p16 = (r << 10) + (g << 4) + lanes
                    pm1 = (r << 10) + (((g << 4) + lanes - 1) & (_TW - 1))
                    for c in range(_C):
                        t = in_v[s, r, wt, c, pl.ds(wl0, _L)]
                        plsc.store_scatter(
                            out_v.at[s],
                            [p16, jnp.full((_L,), c, jnp.int32)], t)
                        plsc.store_scatter(
                            out_v.at[s],
                            [pm1, jnp.full((_L,), c + 4, jnp.int32)], t)

            issue_in(0, 0)

            def blk_body(bi, _):
                s = bi & 1

                @pl.when(bi + 1 < blocks)
                def _():
                    issue_in(bi + 1, 1 - s)

                wait_in(bi, s)

                @pl.when(bi > 1)
                def _():
                    pltpu.make_async_copy(out_v.at[s], tab_slice(bi - 2),
                                          sem_out).wait()

                transpose(s)
                pltpu.async_copy(out_v.at[s], tab_slice(bi), sem_out)
                return ()

            lax.fori_loop(0, blocks, blk_body, ())
            for tail in (blocks - 2, blocks - 1):
                pltpu.make_async_copy(out_v.at[tail & 1], tab_slice(tail),
                                      sem_out).wait()

        for mm, tex_ref in enumerate((t0, t1, t2, t3)):
            @pl.when(m == mm)
            def _(tex_ref=tex_ref, mm=mm):
                build(tex_ref, mm)

    return bk(*tex5s)


def _sc_sample(uv5, fm5, table, b, h, w):
    n = b * h * w
    info = plsc.get_sparse_core_info()
    nw = info.num_cores * info.num_subcores
    per_w = n // nw
    rows_w = per_w // w
    rpc = _P // w
    n_chunks = per_w // _P
    nwt = w // _WL
    mesh = plsc.VectorSubcoreMesh(core_axis_name="c", subcore_axis_name="s")

    @functools.partial(
        pl.kernel,
        mesh=mesh,
        out_type=jax.ShapeDtypeStruct((b, h, nwt, _C, _WL), jnp.float32),
        compiler_params=pltpu.CompilerParams(
            needs_layout_passes=False, use_tc_tiling_on_sc=False),
        scratch_types=[
            pltpu.VMEM((3, rpc, nwt, 2, _WL), jnp.float32),
            pltpu.VMEM((3, nwt, rpc, _WL), jnp.int32),
            pltpu.VMEM((2, 2, _P), jnp.int32),
            pltpu.VMEM((2, _P, 2 * _C), jnp.float32),
            pltpu.VMEM((2, _P, 2 * _C), jnp.float32),
            pltpu.VMEM((2, rpc, nwt, _C, _WL), jnp.float32),
            pltpu.SemaphoreType.DMA,
            pltpu.SemaphoreType.DMA,
            pltpu.SemaphoreType.DMA,
        ],
    )
    def k(uv_hbm, fm_hbm, tab_hbm, out_hbm,
          uv_v, fm_v, idx_v, top_v, bot_v, o_v, sem_in, sem_g, sem_out):
        wid = lax.axis_index("s") * info.num_cores + lax.axis_index("c")
        lanes = lax.iota(jnp.int32, _L)

        def frac(val, scale):
            x = val * scale - 0.5
            xt = x.astype(jnp.int32)
            x0 = jnp.where(x < xt.astype(jnp.float32), xt - 1, xt)
            return x0, x - x0.astype(jnp.float32)

        def coords(kc):
            r0 = wid * rows_w + kc * rpc
            return r0 // h, r0 % h

        def issue_in(kc, s):
            bi, h0 = coords(kc)
            pltpu.async_copy(uv_hbm.at[bi, pl.ds(h0, rpc)], uv_v.at[s],
                             sem_in)
            pltpu.async_copy(fm_hbm.at[bi, h0 // 8, :, pl.ds(h0 % 8, rpc)],
                             fm_v.at[s], sem_in)

        def wait_in(kc, s):
            bi, h0 = coords(kc)
            pltpu.make_async_copy(uv_hbm.at[bi, pl.ds(h0, rpc)], uv_v.at[s],
                                  sem_in).wait()
            pltpu.make_async_copy(fm_hbm.at[bi, h0 // 8, :,
                                            pl.ds(h0 % 8, rpc)],
                                  fm_v.at[s], sem_in).wait()

        def phase2_and_fire(s3, s):
            @plsc.parallel_loop(0, _P // _L, unroll=4)
            def _(i):
                hs = i >> 5
                wt = (i >> 3) & 3
                wl0 = (i & 7) << 4
                csl = pl.ds(wl0, _L)
                uu = uv_v[s3, hs, wt, 0, csl]
                vv = uv_v[s3, hs, wt, 1, csl]
                fm = fm_v[s3, wt, hs, csl]
                x0, _fx = frac(uu, float(_TW))
                y0, _fy = frac(vv, float(_TH))
                base_m = (fm << 20) + (x0 & (_TW - 1))
                sl = pl.ds(i * _L, _L)
                idx_v[s, 0, sl] = base_m + ((y0 & (_TH - 1)) << 10)
                idx_v[s, 1, sl] = base_m + (((y0 + 1) & (_TH - 1)) << 10)

            pltpu.async_copy(tab_hbm.at[idx_v.at[s, 0]], top_v.at[s], sem_g)
            pltpu.async_copy(tab_hbm.at[idx_v.at[s, 1]], bot_v.at[s], sem_g)

        def combine_and_out(kc, s):
            s3 = kc % 3
            pltpu.make_async_copy(tab_hbm.at[idx_v.at[s, 0]], top_v.at[s],
                                  sem_g).wait()
            pltpu.make_async_copy(tab_hbm.at[idx_v.at[s, 1]], bot_v.at[s],
                                  sem_g).wait()

            @pl.when(kc > 1)
            def _():
                bi2, h02 = coords(kc - 2)
                pltpu.make_async_copy(o_v.at[s], out_hbm.at[bi2,
                                                            pl.ds(h02, rpc)],
                                      sem_out).wait()

            @plsc.parallel_loop(0, _P // _L, unroll=2)
            def _(i):
                hs = i >> 5
                wt = (i >> 3) & 3
                wl0 = (i & 7) << 4
                csl = pl.ds(wl0, _L)
                uu = uv_v[s3, hs, wt, 0, csl]
                vv = uv_v[s3, hs, wt, 1, csl]
                _x0, fx = frac(uu, float(_TW))
                _y0, fy = frac(vv, float(_TH))
                omx = 1.0 - fx
                omy = 1.0 - fy
                p16 = i * _L + lanes
                for c in range(_C):
                    cc = jnp.full((_L,), c, jnp.int32)
                    cc1 = jnp.full((_L,), c + 4, jnp.int32)
                    t00 = plsc.load_gather(top_v.at[s], [p16, cc])
                    t01 = plsc.load_gather(top_v.at[s], [p16, cc1])
                    t10 = plsc.load_gather(bot_v.at[s], [p16, cc])
                    t11 = plsc.load_gather(bot_v.at[s], [p16, cc1])
                    top = t00 * omx + t01 * fx
                    bot = t10 * omx + t11 * fx
                    o_v[s, hs, wt, c, csl] = top * omy + bot * fy

            bi, h0 = coords(kc)
            pltpu.async_copy(o_v.at[s], out_hbm.at[bi, pl.ds(h0, rpc)],
                             sem_out)

        issue_in(0, 0)

        def chunk_body(kc, _):
            s = kc & 1
            s3 = kc % 3
            wait_in(kc, s3)

            @pl.when(kc + 1 < n_chunks)
            def _():
                issue_in(kc + 1, (kc + 1) % 3)

            phase2_and_fire(s3, s)

            @pl.when(kc > 0)
            def _():
                combine_and_out(kc - 1, 1 - s)
            return ()

        lax.fori_loop(0, n_chunks, chunk_body, ())
        combine_and_out(n_chunks - 1, (n_chunks - 1) & 1)
        for tail in (n_chunks - 2, n_chunks - 1):
            bi, h0 = coords(tail)
            pltpu.make_async_copy(o_v.at[tail & 1],
                                  out_hbm.at[bi, pl.ds(h0, rpc)],
                                  sem_out).wait()

    return k(uv5, fm5, table)


def kernel(uv, f_mat, tex0, tex1, tex2, tex3):
    b, h, w, _ = uv.shape
    tex5s = [
        t.reshape(_TH, _TW // _WL, _WL, _C).transpose(0, 1, 3, 2)
        for t in (tex0, tex1, tex2, tex3)
    ]
    table = _sc_build_table(tex5s)
    uv5 = uv.reshape(b, h, w // _WL, _WL, 2).transpose(0, 1, 2, 4, 3)
    fm5 = f_mat.reshape(b, h // 8, 8, w // _WL, _WL).transpose(0, 1, 3, 2, 4)
    out5 = _sc_sample(uv5, fm5, table, b, h, w)
    return out5.transpose(0, 1, 2, 4, 3).reshape(b, h, w, _C)
